# baseline (speedup 1.0000x reference)
import functools

import jax
import jax.numpy as jnp
from jax import lax
from jax.experimental import pallas as pl
from jax.experimental.pallas import tpu as pltpu
from jax.experimental.pallas import tpu_sc as plsc

_N = 10000
_E = 160000
_D = 256
_H = 256
_EPS = 1e-5

_RB = 400
_NRB = _N // _RB

_NSUB = 16
_B = 64
_NBUF = 4
_NB = 160
_CH = _NB * _B
_EPAD = _NSUB * _CH
_RPT = 632
_ACC_ROWS = _NSUB * _RPT
_GARBAGE = 10100


def _mm_stats_body(x_ref, w_ref, h1_ref, stats_ref, acc_ref):
    i = pl.program_id(0)
    h1 = jnp.dot(x_ref[...], w_ref[...], preferred_element_type=jnp.float32)
    h1_ref[...] = h1
    s = jnp.sum(h1, axis=0, keepdims=True)
    s2 = jnp.sum(h1 * h1, axis=0, keepdims=True)
    ss = jnp.concatenate([s, s2], axis=0)

    @pl.when(i == 0)
    def _():
        acc_ref[...] = jnp.zeros_like(acc_ref)

    acc_ref[...] += ss

    @pl.when(i == pl.num_programs(0) - 1)
    def _():
        stats_ref[...] = acc_ref[...]


_mm_stats = pl.pallas_call(
    _mm_stats_body,
    grid=(_NRB,),
    in_specs=[
        pl.BlockSpec((_RB, _D), lambda i: (i, 0)),
        pl.BlockSpec((_D, _H), lambda i: (0, 0)),
    ],
    out_specs=[
        pl.BlockSpec((_RB, _H), lambda i: (i, 0)),
        pl.BlockSpec((2, _H), lambda i: (0, 0)),
    ],
    out_shape=[
        jax.ShapeDtypeStruct((_N, _H), jnp.float32),
        jax.ShapeDtypeStruct((2, _H), jnp.float32),
    ],
    scratch_shapes=[pltpu.VMEM((2, _H), jnp.float32)],
)


def _mm2_body(h1_ref, st_ref, g1_ref, be1_ref, w_ref, z_ref):
    mean = st_ref[0:1, :] * (1.0 / _N)
    var = st_ref[1:2, :] * (1.0 / _N) - mean * mean
    a = g1_ref[...] * lax.rsqrt(var + _EPS)
    b = be1_ref[...] - mean * a
    h1s = h1_ref[...] * a + b
    z = jnp.dot(h1s, w_ref[...], preferred_element_type=jnp.float32)
    z_ref[0] = z[:, 0:128]
    z_ref[1] = z[:, 128:256]


_mm2 = pl.pallas_call(
    _mm2_body,
    grid=(_NRB,),
    in_specs=[
        pl.BlockSpec((_RB, _H), lambda i: (i, 0)),
        pl.BlockSpec((2, _H), lambda i: (0, 0)),
        pl.BlockSpec((1, _H), lambda i: (0, 0)),
        pl.BlockSpec((1, _H), lambda i: (0, 0)),
        pl.BlockSpec((_H, _D), lambda i: (0, 0)),
    ],
    out_specs=pl.BlockSpec((2, _RB, 128), lambda i: (0, i, 0)),
    out_shape=jax.ShapeDtypeStruct((2, _N, 128), jnp.float32),
)


def _stats2_body(h_ref, stats_ref, acc_ref):
    i = pl.program_id(0)
    hb = h_ref[...]
    s = jnp.sum(hb, axis=1)
    s2 = jnp.sum(hb * hb, axis=1)
    ss = jnp.stack([s, s2], axis=0)

    @pl.when(i == 0)
    def _():
        acc_ref[...] = jnp.zeros_like(acc_ref)

    acc_ref[...] += ss

    @pl.when(i == pl.num_programs(0) - 1)
    def _():
        stats_ref[...] = acc_ref[...]


_stats2 = pl.pallas_call(
    _stats2_body,
    grid=(_NRB,),
    in_specs=[pl.BlockSpec((2, _RB, 128), lambda i: (0, i, 0))],
    out_specs=pl.BlockSpec((2, 2, 128), lambda i: (0, 0, 0)),
    out_shape=jax.ShapeDtypeStruct((2, 2, 128), jnp.float32),
    scratch_shapes=[pltpu.VMEM((2, 2, 128), jnp.float32)],
)


def _bn2_body(h_ref, st_ref, g_ref, be_ref, out_ref):
    mean = st_ref[0] * (1.0 / _N)
    var = st_ref[1] * (1.0 / _N) - mean * mean
    a = g_ref[...] * lax.rsqrt(var + _EPS)
    b = be_ref[...] - mean * a
    hb = h_ref[...]
    y = hb * a[:, None, :] + b[:, None, :]
    out_ref[:, 0:128] = y[0]
    out_ref[:, 128:256] = y[1]


_bn2 = pl.pallas_call(
    _bn2_body,
    grid=(_NRB,),
    in_specs=[
        pl.BlockSpec((2, _RB, 128), lambda i: (0, i, 0)),
        pl.BlockSpec((2, 2, 128), lambda i: (0, 0, 0)),
        pl.BlockSpec((2, 128), lambda i: (0, 0)),
        pl.BlockSpec((2, 128), lambda i: (0, 0)),
    ],
    out_specs=pl.BlockSpec((_RB, _D), lambda i: (i, 0)),
    out_shape=jax.ShapeDtypeStruct((_N, _D), jnp.float32),
)


def _seg_body(zf_h, src0_h, src1_h, dst_h, zrows_h, out_h,
              src_v, dring, rb0, rb1, rb2, rb3, acc_s,
              g0, g1, g2, g3, s0, s1, s2, s3, d0, d1, d2, d3):
    c = lax.axis_index("c")
    s = lax.axis_index("s")
    r0 = s * _RPT
    rows = (rb0, rb1, rb2, rb3)
    gsem = (g0, g1, g2, g3)
    ssem = (s0, s1, s2, s3)
    dsem = (d0, d1, d2, d3)

    pltpu.sync_copy(zrows_h, acc_s.at[pl.ds(r0, _RPT)])

    base = s * _CH

    @pl.when(c == 0)
    def _():
        pltpu.sync_copy(src0_h.at[pl.ds(base, _CH)], src_v)

    @pl.when(c == 1)
    def _():
        pltpu.sync_copy(src1_h.at[pl.ds(base, _CH)], src_v)

    def _didx(j, b):
        return pltpu.make_async_copy(
            dst_h.at[s, pl.ds(j, 1)], dring.at[pl.ds(b, 1)], dsem[b])

    def _gather(j, b):
        off = pl.multiple_of(j * _B, _B)
        return pltpu.make_async_copy(
            zf_h.at[src_v.at[pl.ds(off, _B)]], rows[b], gsem[b])

    def _scatter(b):
        return pltpu.make_async_copy(rows[b], acc_s.at[dring.at[b]], ssem[b])

    for b in range(_NBUF - 1):
        _didx(b, b).start()
        _gather(b, b).start()
    plsc.subcore_barrier()

    def body(j4, carry):
        j0 = j4 * _NBUF
        for b in range(_NBUF):
            j = j0 + b
            _gather(j, b).wait()
            _didx(j, b).wait()
            _scatter(b).start(add=True)
            jn = j + _NBUF - 1
            bn = (b + _NBUF - 1) % _NBUF

            @pl.when((jn < _NB) & (j >= 1))
            def _():
                _scatter(bn).wait()

            @pl.when(jn < _NB)
            def _():
                _didx(jn, bn).start()
                _gather(jn, bn).start()

        return carry

    lax.fori_loop(0, _NB // _NBUF, body, 0)
    for b in range(_NBUF):
        _scatter(b).wait()
    plsc.subcore_barrier()

    out_base = c * _N + r0

    @pl.when(s < _NSUB - 1)
    def _():
        pltpu.sync_copy(acc_s.at[pl.ds(r0, _RPT)], out_h.at[pl.ds(out_base, _RPT)])

    @pl.when(s == _NSUB - 1)
    def _():
        pltpu.sync_copy(acc_s.at[pl.ds(r0, _N - (_NSUB - 1) * _RPT)],
                        out_h.at[pl.ds(out_base, _N - (_NSUB - 1) * _RPT)])


_seg_sum = functools.partial(
    pl.kernel,
    mesh=plsc.VectorSubcoreMesh(core_axis_name="c", subcore_axis_name="s"),
    out_type=jax.ShapeDtypeStruct((2 * _N, 128), jnp.float32),
    scratch_types=(
        [pltpu.VMEM((_CH,), jnp.int32), pltpu.VMEM((_NBUF, _B), jnp.int32)]
        + [pltpu.VMEM((_B, 128), jnp.float32) for _ in range(_NBUF)]
        + [pltpu.VMEM_SHARED((_ACC_ROWS, 128), jnp.float32)]
        + [pltpu.SemaphoreType.DMA for _ in range(3 * _NBUF)]
    ),
)(_seg_body)


def kernel(x, edge_index, W1, bn1_gamma, bn1_beta, W2, bn2_gamma, bn2_beta):
    h1, st1 = _mm_stats(x, W1)
    z = _mm2(h1, st1, bn1_gamma.reshape(1, _H), bn1_beta.reshape(1, _H), W2)

    src = edge_index[0]
    dst = edge_index[1]
    pad = _EPAD - _E
    src_p = jnp.concatenate([src, jnp.zeros((pad,), jnp.int32)])
    dst_p = jnp.concatenate([dst, jnp.full((pad,), _GARBAGE, jnp.int32)])
    dst_p = dst_p.reshape(_NSUB, _NB, _B)
    zf = z.reshape(2 * _N, 128)
    zrows = jnp.zeros((_RPT, 128), jnp.float32)

    hf = _seg_sum(zf, src_p, src_p + _N, dst_p, zrows)
    h2 = hf.reshape(2, _N, 128)

    st2 = _stats2(h2)
    return _bn2(h2, st2, bn2_gamma.reshape(2, 128), bn2_beta.reshape(2, 128))


# speedup vs baseline: 1.2757x; 1.2757x over previous
import functools

import jax
import jax.numpy as jnp
from jax import lax
from jax.experimental import pallas as pl
from jax.experimental.pallas import tpu as pltpu
from jax.experimental.pallas import tpu_sc as plsc

_N = 10000
_E = 160000
_D = 256
_H = 256
_EPS = 1e-5

_RB = 400
_NRB = _N // _RB

_NSUB = 16
_B = 64
_NBUF = 4
_NB = 160
_CH = _NB * _B
_EPAD = _NSUB * _CH
_RPT = 632
_ACC_ROWS = _NSUB * _RPT
_GARBAGE = 10100


def _mm_stats_body(x_ref, w_ref, h1_ref, stats_ref, acc_ref):
    i = pl.program_id(0)
    h1 = jnp.dot(x_ref[...], w_ref[...], preferred_element_type=jnp.float32)
    h1_ref[...] = h1
    s = jnp.sum(h1, axis=0, keepdims=True)
    s2 = jnp.sum(h1 * h1, axis=0, keepdims=True)
    ss = jnp.concatenate([s, s2], axis=0)

    @pl.when(i == 0)
    def _():
        acc_ref[...] = jnp.zeros_like(acc_ref)

    acc_ref[...] += ss

    @pl.when(i == pl.num_programs(0) - 1)
    def _():
        stats_ref[...] = acc_ref[...]


_mm_stats = pl.pallas_call(
    _mm_stats_body,
    grid=(_NRB,),
    in_specs=[
        pl.BlockSpec((_RB, _D), lambda i: (i, 0)),
        pl.BlockSpec((_D, _H), lambda i: (0, 0)),
    ],
    out_specs=[
        pl.BlockSpec((_RB, _H), lambda i: (i, 0)),
        pl.BlockSpec((2, _H), lambda i: (0, 0)),
    ],
    out_shape=[
        jax.ShapeDtypeStruct((_N, _H), jnp.float32),
        jax.ShapeDtypeStruct((2, _H), jnp.float32),
    ],
    scratch_shapes=[pltpu.VMEM((2, _H), jnp.float32)],
)


def _mm2_body(h1_ref, ab_ref, w_ref, z_ref):
    h1s = h1_ref[...] * ab_ref[0:1, :] + ab_ref[1:2, :]
    z = jnp.dot(h1s, w_ref[...], preferred_element_type=jnp.float32)
    z_ref[0] = z[:, 0:128]
    z_ref[1] = z[:, 128:256]


_mm2 = pl.pallas_call(
    _mm2_body,
    grid=(_NRB,),
    in_specs=[
        pl.BlockSpec((_RB, _H), lambda i: (i, 0)),
        pl.BlockSpec((2, _H), lambda i: (0, 0)),
        pl.BlockSpec((_H, _D), lambda i: (0, 0)),
    ],
    out_specs=pl.BlockSpec((2, _RB, 128), lambda i: (0, i, 0)),
    out_shape=jax.ShapeDtypeStruct((2, _N, 128), jnp.float32),
)


def _stats2_body(h_ref, stats_ref, acc_ref):
    i = pl.program_id(0)
    hb = h_ref[...]
    s = jnp.sum(hb, axis=1)
    s2 = jnp.sum(hb * hb, axis=1)
    ss = jnp.stack([s, s2], axis=0)

    @pl.when(i == 0)
    def _():
        acc_ref[...] = jnp.zeros_like(acc_ref)

    acc_ref[...] += ss

    @pl.when(i == pl.num_programs(0) - 1)
    def _():
        stats_ref[...] = acc_ref[...]


_stats2 = pl.pallas_call(
    _stats2_body,
    grid=(_NRB,),
    in_specs=[pl.BlockSpec((2, _RB, 128), lambda i: (0, i, 0))],
    out_specs=pl.BlockSpec((2, 2, 128), lambda i: (0, 0, 0)),
    out_shape=jax.ShapeDtypeStruct((2, 2, 128), jnp.float32),
    scratch_shapes=[pltpu.VMEM((2, 2, 128), jnp.float32)],
)


def _bn2_body(h_ref, ab_ref, out_ref):
    hb = h_ref[...]
    a = ab_ref[0]
    b = ab_ref[1]
    y = hb * a[:, None, :] + b[:, None, :]
    out_ref[:, 0:128] = y[0]
    out_ref[:, 128:256] = y[1]


_bn2 = pl.pallas_call(
    _bn2_body,
    grid=(_NRB,),
    in_specs=[
        pl.BlockSpec((2, _RB, 128), lambda i: (0, i, 0)),
        pl.BlockSpec((2, 2, 128), lambda i: (0, 0, 0)),
    ],
    out_specs=pl.BlockSpec((_RB, _D), lambda i: (i, 0)),
    out_shape=jax.ShapeDtypeStruct((_N, _D), jnp.float32),
)


def _seg_body(zf_h, src0_h, src1_h, dst_h, zrows_h, out_h,
              src_v, dring, rb0, rb1, rb2, rb3, acc_s,
              g0, g1, g2, g3, s0, s1, s2, s3, d0, d1, d2, d3):
    c = lax.axis_index("c")
    s = lax.axis_index("s")
    r0 = s * _RPT
    rows = (rb0, rb1, rb2, rb3)
    gsem = (g0, g1, g2, g3)
    ssem = (s0, s1, s2, s3)
    dsem = (d0, d1, d2, d3)

    pltpu.sync_copy(zrows_h, acc_s.at[pl.ds(r0, _RPT)])

    base = s * _CH

    @pl.when(c == 0)
    def _():
        pltpu.sync_copy(src0_h.at[pl.ds(base, _CH)], src_v)

    @pl.when(c == 1)
    def _():
        pltpu.sync_copy(src1_h.at[pl.ds(base, _CH)], src_v)

    def _didx(j, b):
        return pltpu.make_async_copy(
            dst_h.at[s, pl.ds(j, 1)], dring.at[pl.ds(b, 1)], dsem[b])

    def _gather(j, b):
        off = pl.multiple_of(j * _B, _B)
        return pltpu.make_async_copy(
            zf_h.at[src_v.at[pl.ds(off, _B)]], rows[b], gsem[b])

    def _scatter(b):
        return pltpu.make_async_copy(rows[b], acc_s.at[dring.at[b]], ssem[b])

    for b in range(_NBUF - 1):
        _didx(b, b).start()
        _gather(b, b).start()
    plsc.subcore_barrier()

    def body(j4, carry):
        j0 = j4 * _NBUF
        for b in range(_NBUF):
            j = j0 + b
            _gather(j, b).wait()
            _didx(j, b).wait()
            _scatter(b).start(add=True)
            jn = j + _NBUF - 1
            bn = (b + _NBUF - 1) % _NBUF

            @pl.when((jn < _NB) & (j >= 1))
            def _():
                _scatter(bn).wait()

            @pl.when(jn < _NB)
            def _():
                _didx(jn, bn).start()
                _gather(jn, bn).start()

        return carry

    lax.fori_loop(0, _NB // _NBUF, body, 0)
    for b in range(_NBUF):
        _scatter(b).wait()
    plsc.subcore_barrier()

    out_base = c * _N + r0

    @pl.when(s < _NSUB - 1)
    def _():
        pltpu.sync_copy(acc_s.at[pl.ds(r0, _RPT)], out_h.at[pl.ds(out_base, _RPT)])

    @pl.when(s == _NSUB - 1)
    def _():
        pltpu.sync_copy(acc_s.at[pl.ds(r0, _N - (_NSUB - 1) * _RPT)],
                        out_h.at[pl.ds(out_base, _N - (_NSUB - 1) * _RPT)])


_seg_sum = functools.partial(
    pl.kernel,
    mesh=plsc.VectorSubcoreMesh(core_axis_name="c", subcore_axis_name="s"),
    out_type=jax.ShapeDtypeStruct((2 * _N, 128), jnp.float32),
    scratch_types=(
        [pltpu.VMEM((_CH,), jnp.int32), pltpu.VMEM((_NBUF, _B), jnp.int32)]
        + [pltpu.VMEM((_B, 128), jnp.float32) for _ in range(_NBUF)]
        + [pltpu.VMEM_SHARED((_ACC_ROWS, 128), jnp.float32)]
        + [pltpu.SemaphoreType.DMA for _ in range(3 * _NBUF)]
    ),
)(_seg_body)


def kernel(x, edge_index, W1, bn1_gamma, bn1_beta, W2, bn2_gamma, bn2_beta):
    h1, st1 = _mm_stats(x, W1)
    mean1 = st1[0] / _N
    var1 = st1[1] / _N - mean1 * mean1
    a1 = bn1_gamma / jnp.sqrt(var1 + _EPS)
    b1 = bn1_beta - mean1 * a1
    ab1 = jnp.stack([a1, b1])

    z = _mm2(h1, ab1, W2)

    src = edge_index[0]
    dst = edge_index[1]
    pad = _EPAD - _E
    src_p = jnp.concatenate([src, jnp.zeros((pad,), jnp.int32)])
    dst_p = jnp.concatenate([dst, jnp.full((pad,), _GARBAGE, jnp.int32)])
    dst_p = dst_p.reshape(_NSUB, _NB, _B)
    zf = z.reshape(2 * _N, 128)
    zrows = jnp.zeros((_RPT, 128), jnp.float32)

    hf = _seg_sum(zf, src_p, src_p + _N, dst_p, zrows)
    h2 = hf.reshape(2, _N, 128)

    st2 = _stats2(h2)
    mean2 = st2[0] / _N
    var2 = st2[1] / _N - mean2 * mean2
    a2 = bn2_gamma.reshape(2, 128) / jnp.sqrt(var2 + _EPS)
    b2 = bn2_beta.reshape(2, 128) - mean2 * a2
    ab2 = jnp.stack([a2, b2])

    return _bn2(h2, ab2)
